# baseline (speedup 1.0000x reference)
import functools

import jax
import jax.numpy as jnp
from jax import lax
from jax.experimental import pallas as pl
from jax.experimental.pallas import tpu as pltpu
from jax.experimental.pallas import tpu_sc as plsc

NUM_CLASSES = 1000
FEAT_DIM = 128
BATCH = 4096

NC = 2
NS = 16
L = 16
NW = NC * NS
BPW = BATCH // NW
CHUNKS = FEAT_DIM // L
NBUF = 1
RPC = BPW // NBUF


def _body(y_hbm, feat_hbm, centers_hbm, out_hbm, idx_v, cen_v, feat_v,
          stage_v, gsem):
    cid = lax.axis_index("c")
    sid = lax.axis_index("s")
    wid = sid * NC + cid
    base = wid * BPW

    pltpu.sync_copy(y_hbm.at[pl.ds(base, BPW)], idx_v)
    gather = pltpu.async_copy(centers_hbm.at[idx_v], cen_v, gsem)
    pltpu.sync_copy(feat_hbm.at[pl.ds(base, BPW)], feat_v)
    gather.wait()

    def row(r, a):
        new = []
        for c in range(CHUNKS):
            f = feat_v[r, pl.ds(c * L, L)]
            g = cen_v[r, pl.ds(c * L, L)]
            d = f - g
            new.append(a[c] + d * d)
        return tuple(new)

    accs = plsc.parallel_loop(
        0, BPW,
        carry=tuple(jnp.zeros((L,), jnp.float32) for _ in range(CHUNKS)))(row)

    acc = ((accs[0] + accs[1]) + (accs[2] + accs[3])) + \
          ((accs[4] + accs[5]) + (accs[6] + accs[7]))
    stage_v[...] = acc
    pltpu.sync_copy(stage_v, out_hbm.at[wid])


@functools.partial(jax.jit, static_argnames=())
def kernel(y, feat, centers):
    mesh = plsc.VectorSubcoreMesh(
        core_axis_name="c", subcore_axis_name="s",
        num_cores=NC, num_subcores=NS)
    partials = pl.kernel(
        _body,
        out_type=jax.ShapeDtypeStruct((NW, L), jnp.float32),
        mesh=mesh,
        scratch_types=[
            pltpu.VMEM((BPW,), jnp.int32),
            pltpu.VMEM((BPW, FEAT_DIM), jnp.float32),
            pltpu.VMEM((BPW, FEAT_DIM), jnp.float32),
            pltpu.VMEM((L,), jnp.float32),
            pltpu.SemaphoreType.DMA,
        ],
    )(y, feat, centers)
    return jnp.sum(partials) * jnp.float32(0.5)


# speedup vs baseline: 1.0134x; 1.0134x over previous
import functools

import jax
import jax.numpy as jnp
from jax import lax
from jax.experimental import pallas as pl
from jax.experimental.pallas import tpu as pltpu
from jax.experimental.pallas import tpu_sc as plsc

NUM_CLASSES = 1000
FEAT_DIM = 128
BATCH = 4096

NC = 2
NS = 16
L = 16
NW = NC * NS
BPW = BATCH // NW
CHUNKS = FEAT_DIM // L
NBUF = 1
RPC = BPW // NBUF


def _body(y_hbm, feat_hbm, centers_hbm, out_hbm, idx_v, cen_v, feat_v,
          stage_v, fsem, gsems):
    cid = lax.axis_index("c")
    sid = lax.axis_index("s")
    wid = sid * NC + cid
    base = wid * BPW

    feat_cp = pltpu.async_copy(feat_hbm.at[pl.ds(base, BPW)], feat_v, fsem)
    pltpu.sync_copy(y_hbm.at[pl.ds(base, BPW)], idx_v)
    gathers = [
        pltpu.async_copy(
            centers_hbm.at[idx_v.at[pl.ds(k * RPC, RPC)]],
            cen_v.at[pl.ds(k * RPC, RPC)], gsems.at[k])
        for k in range(NBUF)
    ]
    feat_cp.wait()

    accs = tuple(jnp.zeros((L,), jnp.float32) for _ in range(CHUNKS))
    for k in range(NBUF):
        gathers[k].wait()

        def row(r, a):
            new = []
            for c in range(CHUNKS):
                f = feat_v[r, pl.ds(c * L, L)]
                g = cen_v[r, pl.ds(c * L, L)]
                d = f - g
                new.append(a[c] + d * d)
            return tuple(new)

        accs = plsc.parallel_loop(k * RPC, (k + 1) * RPC, carry=accs)(row)

    acc = ((accs[0] + accs[1]) + (accs[2] + accs[3])) + \
          ((accs[4] + accs[5]) + (accs[6] + accs[7]))
    stage_v[...] = acc
    pltpu.sync_copy(stage_v, out_hbm.at[wid])


@functools.partial(jax.jit, static_argnames=())
def kernel(y, feat, centers):
    mesh = plsc.VectorSubcoreMesh(
        core_axis_name="c", subcore_axis_name="s",
        num_cores=NC, num_subcores=NS)
    partials = pl.kernel(
        _body,
        out_type=jax.ShapeDtypeStruct((NW, L), jnp.float32),
        mesh=mesh,
        scratch_types=[
            pltpu.VMEM((BPW,), jnp.int32),
            pltpu.VMEM((BPW, FEAT_DIM), jnp.float32),
            pltpu.VMEM((BPW, FEAT_DIM), jnp.float32),
            pltpu.VMEM((L,), jnp.float32),
            pltpu.SemaphoreType.DMA,
            pltpu.SemaphoreType.DMA((NBUF,)),
        ],
    )(y, feat, centers)
    return jnp.sum(partials) * jnp.float32(0.5)
